# Initial kernel scaffold; baseline (speedup 1.0000x reference)
import jax
import jax.numpy as jnp
from jax.experimental import pallas as pl


def kernel(h, pos, edge_index, edge_type, node_type, emb, W1, b1, W2, b2, U1, c1, U2, c2, gamma_ln, beta_ln):
    raise NotImplementedError("write your pallas kernel here")


# speedup vs baseline: 1.6693x; 1.6693x over previous
import functools

import jax
import jax.numpy as jnp
from jax import lax
from jax.experimental import pallas as pl
from jax.experimental.pallas import tpu as pltpu

N = 10000
E = 320000
D = 128
NUM_RBF = 32
CUTOFF = 6.0
PAD = 16
TW = D + PAD
STEP = CUTOFF / (NUM_RBF - 1)
GAMMA = 1.0 / max(STEP * STEP, 1e-06)

BN = 1000
BE = 2560


def _silu(x):
    return x * (1.0 / (1.0 + jnp.exp(-x)))


def _table_body(h_ref, posp_ref, w1at_ref, w1bt_ref, a_ref, b_ref):
    h = h_ref[...]
    posp = posp_ref[...]
    a_ref[:, :D] = jnp.dot(h, w1at_ref[...], preferred_element_type=jnp.float32)
    a_ref[:, D:TW] = posp
    b_ref[:, :D] = jnp.dot(h, w1bt_ref[...], preferred_element_type=jnp.float32)
    b_ref[:, D:TW] = posp


def _build_tables(h, posp, w1at, w1bt):
    grid = N // BN
    return pl.pallas_call(
        _table_body,
        grid=(grid,),
        in_specs=[
            pl.BlockSpec((BN, D), lambda i: (i, 0)),
            pl.BlockSpec((BN, PAD), lambda i: (i, 0)),
            pl.BlockSpec((D, D), lambda i: (0, 0)),
            pl.BlockSpec((D, D), lambda i: (0, 0)),
        ],
        out_specs=[
            pl.BlockSpec((BN, TW), lambda i: (i, 0)),
            pl.BlockSpec((BN, TW), lambda i: (i, 0)),
        ],
        out_shape=[
            jax.ShapeDtypeStruct((N, TW), jnp.float32),
            jax.ShapeDtypeStruct((N, TW), jnp.float32),
        ],
    )(h, posp, w1at, w1bt)


def _edge_body(ga_ref, gb_ref, et_ref, emb_ref, w1ct_ref, w1rt_ref, w1d_ref,
               b1_ref, w2t_ref, b2_ref, out_ref):
    ga = ga_ref[...]
    gb = gb_ref[...]
    rel = ga[:, D:D + 3] - gb[:, D:D + 3]
    dist2 = jnp.sum(rel * rel, axis=1, keepdims=True)
    dist = jnp.sqrt(dist2)
    centers = STEP * lax.broadcasted_iota(jnp.int32, (1, NUM_RBF), 1).astype(jnp.float32)
    diff = dist - centers
    radial = jnp.exp(-GAMMA * diff * diff)
    t = jnp.dot(emb_ref[...], w1ct_ref[...], preferred_element_type=jnp.float32) \
        + b1_ref[...]
    et = et_ref[...]
    tsel = t[0:1, :] * (1.0 - et) + t[1:2, :] * et
    pre1 = (ga[:, :D] + gb[:, :D] + tsel
            + jnp.dot(radial, w1rt_ref[...], preferred_element_type=jnp.float32)
            + dist * w1d_ref[...])
    x = _silu(pre1)
    msg = _silu(jnp.dot(x, w2t_ref[...], preferred_element_type=jnp.float32)
                + b2_ref[...])
    out_ref[:, :D] = msg
    onescol = (lax.broadcasted_iota(jnp.int32, (BE, PAD), 1) == 0)
    out_ref[:, D:TW] = onescol.astype(jnp.float32)


def _edge_mlp(ga, gb, etf, emb, w1ct, w1rt, w1d, b1, w2t, b2):
    grid = E // BE
    full = lambda i: (0, 0)
    return pl.pallas_call(
        _edge_body,
        grid=(grid,),
        in_specs=[
            pl.BlockSpec((BE, TW), lambda i: (i, 0)),
            pl.BlockSpec((BE, TW), lambda i: (i, 0)),
            pl.BlockSpec((BE, 1), lambda i: (i, 0)),
            pl.BlockSpec((2, D), full),
            pl.BlockSpec((D, D), full),
            pl.BlockSpec((NUM_RBF, D), full),
            pl.BlockSpec((1, D), full),
            pl.BlockSpec((1, D), full),
            pl.BlockSpec((D, D), full),
            pl.BlockSpec((1, D), full),
        ],
        out_specs=pl.BlockSpec((BE, TW), lambda i: (i, 0)),
        out_shape=jax.ShapeDtypeStruct((E, TW), jnp.float32),
    )(ga, gb, etf, emb, w1ct, w1rt, w1d, b1, w2t, b2)


def _node_body(h_ref, s0_ref, s1_ref, nt_ref, u1at_ref, u1bt_ref, c1_ref,
               u2t_ref, c2_ref, g_ref, bta_ref, out_ref):
    h = h_ref[...]
    s = s0_ref[...] + s1_ref[...]
    cnt = jnp.maximum(s[:, D:D + 1], 1.0)
    agg = s[:, :D] / cnt
    u = _silu(jnp.dot(h, u1at_ref[...], preferred_element_type=jnp.float32)
              + jnp.dot(agg, u1bt_ref[...], preferred_element_type=jnp.float32)
              + c1_ref[...])
    upd = jnp.dot(u, u2t_ref[...], preferred_element_type=jnp.float32) + c2_ref[...]
    pre = h + upd
    mu = jnp.mean(pre, axis=1, keepdims=True)
    cent = pre - mu
    var = jnp.mean(cent * cent, axis=1, keepdims=True)
    ln = cent * lax.rsqrt(var + 1e-05) * g_ref[...] + bta_ref[...]
    out_ref[...] = jnp.where(nt_ref[...] == 1.0, ln, h)


def _node_update(h, s0, s1, ntf, u1at, u1bt, c1, u2t, c2, g, b):
    grid = N // BN
    full = lambda i: (0, 0)
    return pl.pallas_call(
        _node_body,
        grid=(grid,),
        in_specs=[
            pl.BlockSpec((BN, D), lambda i: (i, 0)),
            pl.BlockSpec((BN, TW), lambda i: (i, 0)),
            pl.BlockSpec((BN, TW), lambda i: (i, 0)),
            pl.BlockSpec((BN, 1), lambda i: (i, 0)),
            pl.BlockSpec((D, D), full),
            pl.BlockSpec((D, D), full),
            pl.BlockSpec((1, D), full),
            pl.BlockSpec((D, D), full),
            pl.BlockSpec((1, D), full),
            pl.BlockSpec((1, D), full),
            pl.BlockSpec((1, D), full),
        ],
        out_specs=pl.BlockSpec((BN, D), lambda i: (i, 0)),
        out_shape=jax.ShapeDtypeStruct((N, D), jnp.float32),
    )(h, s0, s1, ntf, u1at, u1bt, c1, u2t, c2, g, b)


def kernel(h, pos, edge_index, edge_type, node_type, emb, W1, b1, W2, b2,
           U1, c1, U2, c2, gamma_ln, beta_ln):
    src = edge_index[0]
    dst = edge_index[1]
    w1at = W1[:, :D].T
    w1bt = W1[:, D:2 * D].T
    w1ct = W1[:, 2 * D:3 * D].T
    w1rt = W1[:, 3 * D:3 * D + NUM_RBF].T
    w1d = W1[:, 3 * D + NUM_RBF][None, :]
    b1r = b1[None, :]
    w2t = W2.T
    b2r = b2[None, :]
    u1at = U1[:, :D].T
    u1bt = U1[:, D:].T
    c1r = c1[None, :]
    u2t = U2.T
    c2r = c2[None, :]
    gr = gamma_ln[None, :]
    br = beta_ln[None, :]
    posp = jnp.pad(pos, ((0, 0), (0, PAD - 3)))

    ta, tb = _build_tables(h, posp, w1at, w1bt)

    ga = jnp.take(ta, src, axis=0)
    gb = jnp.take(tb, dst, axis=0)
    etf = edge_type.astype(jnp.float32)[:, None]

    msgp = _edge_mlp(ga, gb, etf, emb, w1ct, w1rt, w1d, b1r, w2t, b2r)

    sums = jax.ops.segment_sum(msgp, dst, num_segments=N)
    zeros = jnp.zeros_like(sums)

    ntf = node_type.astype(jnp.float32)[:, None]
    return _node_update(h, sums, zeros, ntf, u1at, u1bt, c1r, u2t, c2r, gr, br)


# speedup vs baseline: 6.1895x; 3.7078x over previous
import functools

import jax
import jax.numpy as jnp
from jax import lax
from jax.experimental import pallas as pl
from jax.experimental.pallas import tpu as pltpu
from jax.experimental.pallas import tpu_sc as plsc

N = 10000
E = 320000
D = 128
NUM_RBF = 32
CUTOFF = 6.0
STEP = CUTOFF / (NUM_RBF - 1)
GAMMA = 1.0 / max(STEP * STEP, 1e-06)

BN = 1000
BE = 2560

_NC = 2
_NS = 16
_NW = _NC * _NS
_EPW = E // _NW
_L = 16

_GCH = 400
_GSUB = 80
_GNSUB = _GCH // _GSUB
_GNCH = _EPW // _GCH

_SCH = 200
_SSUB = 100
_SNSUB = _SCH // _SSUB
_SNCH = _EPW // _SCH
_RPT = 624
_RTAIL = N - _NS * _RPT


def _silu(x):
    return x * (1.0 / (1.0 + jnp.exp(-x)))


def _table_body(h_ref, w1at_ref, w1bt_ref, a_ref, b_ref):
    h = h_ref[...]
    a_ref[...] = jnp.dot(h, w1at_ref[...], preferred_element_type=jnp.float32)
    b_ref[...] = jnp.dot(h, w1bt_ref[...], preferred_element_type=jnp.float32)


def _build_tables(h, w1at, w1bt):
    grid = N // BN
    return pl.pallas_call(
        _table_body,
        grid=(grid,),
        in_specs=[
            pl.BlockSpec((BN, D), lambda i: (i, 0)),
            pl.BlockSpec((D, D), lambda i: (0, 0)),
            pl.BlockSpec((D, D), lambda i: (0, 0)),
        ],
        out_specs=[
            pl.BlockSpec((BN, D), lambda i: (i, 0)),
            pl.BlockSpec((BN, D), lambda i: (i, 0)),
        ],
        out_shape=[
            jax.ShapeDtypeStruct((N, D), jnp.float32),
            jax.ShapeDtypeStruct((N, D), jnp.float32),
        ],
    )(h, w1at, w1bt)


def _sc_mesh():
    return plsc.VectorSubcoreMesh(core_axis_name="c", subcore_axis_name="s")


def _sc_gather_body(ta, tb, srcr, dstr, pxr, pyr, pzr,
                    ga, gb, d2o, cnto,
                    idxs, idxd, buf, d2b, cntb, px, py, pz, sem):
    cid = lax.axis_index("c")
    sid = lax.axis_index("s")
    wid = sid * _NC + cid
    base = wid * _EPW

    pltpu.sync_copy(pxr, px)
    pltpu.sync_copy(pyr, py)
    pltpu.sync_copy(pzr, pz)

    zero16 = jnp.zeros((_L,), jnp.float32)

    def zinit(r, carry):
        cntb[pl.ds(r * _L, _L)] = zero16
        return carry

    lax.fori_loop(0, N // _L, zinit, 0)

    one16 = jnp.ones((_L,), jnp.float32)

    def chunk(i, carry):
        off = base + i * _GCH
        pltpu.sync_copy(srcr.at[pl.ds(off, _GCH)], idxs)
        pltpu.sync_copy(dstr.at[pl.ds(off, _GCH)], idxd)
        ha = []
        for j in range(_GNSUB):
            sl = pl.ds(j * _GSUB, _GSUB)
            ha.append(pltpu.async_copy(ta.at[idxs.at[sl]], buf.at[sl], sem))

        def dcomp(k, c2):
            s16 = pl.ds(k * _L, _L)
            iv_s = idxs[s16]
            iv_d = idxd[s16]
            dx = plsc.load_gather(px, [iv_s]) - plsc.load_gather(px, [iv_d])
            dy = plsc.load_gather(py, [iv_s]) - plsc.load_gather(py, [iv_d])
            dz = plsc.load_gather(pz, [iv_s]) - plsc.load_gather(pz, [iv_d])
            d2b[s16] = dx * dx + dy * dy + dz * dz
            plsc.addupdate_scatter(cntb, [iv_d], one16)
            return c2

        lax.fori_loop(0, _GCH // _L, dcomp, 0)
        pltpu.sync_copy(d2b, d2o.at[pl.ds(off, _GCH)])

        for h in ha:
            h.wait()
        pltpu.sync_copy(buf, ga.at[pl.ds(off, _GCH)])
        hb = []
        for j in range(_GNSUB):
            sl = pl.ds(j * _GSUB, _GSUB)
            hb.append(pltpu.async_copy(tb.at[idxd.at[sl]], buf.at[sl], sem))
        for h in hb:
            h.wait()
        pltpu.sync_copy(buf, gb.at[pl.ds(off, _GCH)])
        return carry

    lax.fori_loop(0, _GNCH, chunk, 0)
    for blk in range(N // BN):
        pltpu.sync_copy(cntb.at[pl.ds(blk * BN, BN)],
                        cnto.at[pl.ds((blk * _NW + wid) * BN, BN)])


def _sc_gather(ta, tb, src, dst, px, py, pz):
    f = pl.kernel(
        _sc_gather_body,
        out_type=[
            jax.ShapeDtypeStruct((E, D), jnp.float32),
            jax.ShapeDtypeStruct((E, D), jnp.float32),
            jax.ShapeDtypeStruct((E,), jnp.float32),
            jax.ShapeDtypeStruct((N * _NW,), jnp.float32),
        ],
        mesh=_sc_mesh(),
        scratch_types=[
            pltpu.VMEM((_GCH,), jnp.int32),
            pltpu.VMEM((_GCH,), jnp.int32),
            pltpu.VMEM((_GCH, D), jnp.float32),
            pltpu.VMEM((_GCH,), jnp.float32),
            pltpu.VMEM((N,), jnp.float32),
            pltpu.VMEM((N,), jnp.float32),
            pltpu.VMEM((N,), jnp.float32),
            pltpu.VMEM((N,), jnp.float32),
            pltpu.SemaphoreType.DMA,
        ],
        compiler_params=pltpu.CompilerParams(needs_layout_passes=False),
    )
    return f(ta, tb, src, dst, px, py, pz)


def _sc_scatter_body(msgp, dst2, zer, out, shared, msgbuf, idxv, sem):
    cid = lax.axis_index("c")
    sid = lax.axis_index("s")
    wid = sid * _NC + cid
    rows = pl.ds(sid * _RPT, _RPT)
    tail = pl.ds(_NS * _RPT, _RTAIL)
    pltpu.sync_copy(zer.at[rows], shared.at[rows])

    @pl.when(sid == 0)
    def _():
        pltpu.sync_copy(zer.at[tail], shared.at[tail])

    plsc.subcore_barrier()
    base = wid * _EPW

    def chunk(i, carry):
        off = base + i * _SCH
        pltpu.sync_copy(msgp.at[pl.ds(off, _SCH)], msgbuf)
        row0 = wid * (_EPW // _SSUB) + i * _SNSUB
        pltpu.sync_copy(dst2.at[pl.ds(row0, _SNSUB)], idxv)
        hs = []
        for j in range(_SNSUB):
            hs.append(pltpu.async_copy(
                msgbuf.at[pl.ds(j * _SSUB, _SSUB)],
                shared.at[idxv.at[j]], sem, add=True))
        for h in hs:
            h.wait()
        return carry

    lax.fori_loop(0, _SNCH, chunk, 0)
    plsc.subcore_barrier()
    pltpu.sync_copy(shared.at[rows], out.at[cid, rows])

    @pl.when(sid == 0)
    def _():
        pltpu.sync_copy(shared.at[tail], out.at[cid, tail])


def _sc_scatter(msgp, dst2, zer):
    f = pl.kernel(
        _sc_scatter_body,
        out_type=jax.ShapeDtypeStruct((_NC, N, D), jnp.float32),
        mesh=_sc_mesh(),
        scratch_types=[
            pltpu.MemorySpace.VMEM_SHARED((N, D), jnp.float32),
            pltpu.VMEM((_SCH, D), jnp.float32),
            pltpu.VMEM((_SNSUB, _SSUB), jnp.int32),
            pltpu.SemaphoreType.DMA,
        ],
        compiler_params=pltpu.CompilerParams(needs_layout_passes=False),
    )
    return f(msgp, dst2, zer)


def _edge_body(ga_ref, gb_ref, d2_ref, et_ref, emb_ref, w1ct_ref, w1rt_ref,
               w1d_ref, b1_ref, w2t_ref, b2_ref, out_ref):
    ga = ga_ref[...]
    gb = gb_ref[...]
    dist = jnp.sqrt(d2_ref[...])
    centers = STEP * lax.broadcasted_iota(jnp.int32, (1, NUM_RBF), 1).astype(jnp.float32)
    diff = dist - centers
    radial = jnp.exp(-GAMMA * diff * diff)
    t = jnp.dot(emb_ref[...], w1ct_ref[...], preferred_element_type=jnp.float32) \
        + b1_ref[...]
    et = et_ref[...]
    tsel = t[0:1, :] * (1.0 - et) + t[1:2, :] * et
    pre1 = (ga + gb + tsel
            + jnp.dot(radial, w1rt_ref[...], preferred_element_type=jnp.float32)
            + dist * w1d_ref[...])
    x = _silu(pre1)
    out_ref[...] = _silu(
        jnp.dot(x, w2t_ref[...], preferred_element_type=jnp.float32)
        + b2_ref[...])


def _edge_mlp(ga, gb, d2, etf, emb, w1ct, w1rt, w1d, b1, w2t, b2):
    grid = E // BE
    full = lambda i: (0, 0)
    return pl.pallas_call(
        _edge_body,
        grid=(grid,),
        in_specs=[
            pl.BlockSpec((BE, D), lambda i: (i, 0)),
            pl.BlockSpec((BE, D), lambda i: (i, 0)),
            pl.BlockSpec((BE, 1), lambda i: (i, 0)),
            pl.BlockSpec((BE, 1), lambda i: (i, 0)),
            pl.BlockSpec((2, D), full),
            pl.BlockSpec((D, D), full),
            pl.BlockSpec((NUM_RBF, D), full),
            pl.BlockSpec((1, D), full),
            pl.BlockSpec((1, D), full),
            pl.BlockSpec((D, D), full),
            pl.BlockSpec((1, D), full),
        ],
        out_specs=pl.BlockSpec((BE, D), lambda i: (i, 0)),
        out_shape=jax.ShapeDtypeStruct((E, D), jnp.float32),
    )(ga, gb, d2, etf, emb, w1ct, w1rt, w1d, b1, w2t, b2)


def _node_body(h_ref, s0_ref, s1_ref, cnt_ref, nt_ref, u1at_ref, u1bt_ref,
               c1_ref, u2t_ref, c2_ref, g_ref, bta_ref, out_ref):
    h = h_ref[...]
    s = s0_ref[...] + s1_ref[...]
    cnt = jnp.sum(cnt_ref[0], axis=0, keepdims=True)
    recip = 1.0 / jnp.maximum(cnt, 1.0)
    ii = lax.broadcasted_iota(jnp.int32, (BN, BN), 0)
    jj = lax.broadcasted_iota(jnp.int32, (BN, BN), 1)
    dg = jnp.where(ii == jj, recip, 0.0)
    agg = jnp.dot(dg, s, preferred_element_type=jnp.float32)
    u = _silu(jnp.dot(h, u1at_ref[...], preferred_element_type=jnp.float32)
              + jnp.dot(agg, u1bt_ref[...], preferred_element_type=jnp.float32)
              + c1_ref[...])
    upd = jnp.dot(u, u2t_ref[...], preferred_element_type=jnp.float32) + c2_ref[...]
    pre = h + upd
    mu = jnp.mean(pre, axis=1, keepdims=True)
    cent = pre - mu
    var = jnp.mean(cent * cent, axis=1, keepdims=True)
    ln = cent * lax.rsqrt(var + 1e-05) * g_ref[...] + bta_ref[...]
    out_ref[...] = jnp.where(nt_ref[...] == 1.0, ln, h)


def _node_update(h, s0, s1, cnt, ntf, u1at, u1bt, c1, u2t, c2, g, b):
    grid = N // BN
    full = lambda i: (0, 0)
    return pl.pallas_call(
        _node_body,
        grid=(grid,),
        in_specs=[
            pl.BlockSpec((BN, D), lambda i: (i, 0)),
            pl.BlockSpec((BN, D), lambda i: (i, 0)),
            pl.BlockSpec((BN, D), lambda i: (i, 0)),
            pl.BlockSpec((1, _NW, BN), lambda i: (i, 0, 0)),
            pl.BlockSpec((BN, 1), lambda i: (i, 0)),
            pl.BlockSpec((D, D), full),
            pl.BlockSpec((D, D), full),
            pl.BlockSpec((1, D), full),
            pl.BlockSpec((D, D), full),
            pl.BlockSpec((1, D), full),
            pl.BlockSpec((1, D), full),
            pl.BlockSpec((1, D), full),
        ],
        out_specs=pl.BlockSpec((BN, D), lambda i: (i, 0)),
        out_shape=jax.ShapeDtypeStruct((N, D), jnp.float32),
    )(h, s0, s1, cnt, ntf, u1at, u1bt, c1, u2t, c2, g, b)


def kernel(h, pos, edge_index, edge_type, node_type, emb, W1, b1, W2, b2,
           U1, c1, U2, c2, gamma_ln, beta_ln):
    src32 = edge_index[0].astype(jnp.int32)
    dst32 = edge_index[1].astype(jnp.int32)
    w1at = W1[:, :D].T
    w1bt = W1[:, D:2 * D].T
    w1ct = W1[:, 2 * D:3 * D].T
    w1rt = W1[:, 3 * D:3 * D + NUM_RBF].T
    w1d = W1[:, 3 * D + NUM_RBF][None, :]
    b1r = b1[None, :]
    w2t = W2.T
    b2r = b2[None, :]
    u1at = U1[:, :D].T
    u1bt = U1[:, D:].T
    c1r = c1[None, :]
    u2t = U2.T
    c2r = c2[None, :]
    gr = gamma_ln[None, :]
    br = beta_ln[None, :]
    px = pos[:, 0]
    py = pos[:, 1]
    pz = pos[:, 2]

    ta, tb = _build_tables(h, w1at, w1bt)

    ga, gb, d2, cntf = _sc_gather(ta, tb, src32, dst32, px, py, pz)
    cnt = cntf.reshape(N // BN, _NW, BN)

    etf = edge_type.astype(jnp.float32)[:, None]
    msgp = _edge_mlp(ga, gb, d2[:, None], etf, emb, w1ct, w1rt, w1d, b1r,
                     w2t, b2r)

    dst2 = dst32.reshape(E // _SSUB, _SSUB)
    zer = jnp.zeros((N, D), jnp.float32)
    parts = _sc_scatter(msgp, dst2, zer)

    ntf = node_type.astype(jnp.float32)[:, None]
    return _node_update(h, parts[0], parts[1], cnt, ntf, u1at, u1bt, c1r,
                        u2t, c2r, gr, br)
